# baseline (speedup 1.0000x reference)
import jax
import jax.numpy as jnp
from jax.experimental import pallas as pl
from jax.experimental.pallas import tpu as pltpu

H = 128
CDIM = 128
NL = 4
K = 16
B = 4
N = 1024
EPS = 1e-6


def _mm(W, v):
    return jnp.einsum('oi,vin->von', W.astype(jnp.bfloat16),
                      v.astype(jnp.bfloat16),
                      preferred_element_type=jnp.float32)


def _vact(v, D):
    d = _mm(D, v)
    dot = jnp.sum(v * d, axis=0, keepdims=True)
    dsq = jnp.sum(d * d, axis=0, keepdims=True)
    f = jnp.where(dot >= 0.0, 0.0, dot / (dsq + EPS))
    return v - f * d


def _fused_kernel(x_ref, xt_ref, ciW_ref, ciD_ref, lW_ref, lD_ref, gW_ref,
                  gD_ref, coW_ref, zmean_ref, z_ref, d2_ref, acc_ref):
    xb = x_ref[0]
    xt = xt_ref[0]

    sq = jnp.sum(xb * xb, axis=0)
    xb16 = xb.astype(jnp.bfloat16)
    gram = jax.lax.dot_general(xb16, xb16, (((0,), (0,)), ((), ())),
                               preferred_element_type=jnp.float32)
    d2_ref[...] = sq[:, None] + sq[None, :] - 2.0 * gram

    x_dir = xb / jnp.clip(jnp.sqrt(sq)[None, :], 1e-12, None)

    iota_j = jax.lax.broadcasted_iota(jnp.int32, (N, N), 0)
    ciW = ciW_ref[...]
    ciD = ciD_ref[...]
    acc_ref[...] = jnp.zeros((3, H, N), jnp.float32)

    def knn_body(k, _):
        d2 = d2_ref[...]
        m = jnp.min(d2, axis=0, keepdims=True)
        cand = jnp.where(d2 == m, iota_j, N)
        amin = jnp.min(cand, axis=0, keepdims=True)
        sel = iota_j == amin
        d2_ref[...] = jnp.where(sel, jnp.float32(jnp.inf), d2)
        nbr = jnp.concatenate([
            jnp.sum(jnp.where(sel, xt[:, c:c + 1], 0.0), axis=0, keepdims=True)
            for c in range(3)
        ], axis=0)
        rel = nbr - xb
        crs = jnp.stack([
            x_dir[1] * nbr[2] - x_dir[2] * nbr[1],
            x_dir[2] * nbr[0] - x_dir[0] * nbr[2],
            x_dir[0] * nbr[1] - x_dir[1] * nbr[0],
        ], axis=0)
        c16 = crs.astype(jnp.bfloat16).astype(jnp.float32)
        r16 = rel.astype(jnp.bfloat16).astype(jnp.float32)
        x16 = xb.astype(jnp.bfloat16).astype(jnp.float32)
        w16 = ciW.astype(jnp.bfloat16).astype(jnp.float32)
        h = (w16[None, :, 0:1] * c16[:, None, :] +
             w16[None, :, 1:2] * r16[:, None, :] +
             w16[None, :, 2:3] * x16[:, None, :])
        acc_ref[...] += _vact(h, ciD)
        return ()

    jax.lax.fori_loop(0, K, knn_body, (), unroll=16)

    y = acc_ref[...] * (1.0 / K)

    feats = []
    for i in range(NL):
        y = _vact(_mm(lW_ref[i], y), lD_ref[i])
        yg = jnp.mean(y, axis=2, keepdims=True)
        ycat = jnp.concatenate([y, jnp.broadcast_to(yg, y.shape)], axis=1)
        y = _vact(_mm(gW_ref[i], ycat), gD_ref[i])
        feats.append(y)

    z = jnp.concatenate(feats, axis=1)
    z = _mm(coW_ref[...], z)
    z_ref[0] = z
    zmean_ref[0] = jnp.mean(z, axis=2)


@jax.jit
def kernel(x, conv_in_W, conv_in_D, layer0_W, layer0_D, layer1_W, layer1_D,
           layer2_W, layer2_D, layer3_W, layer3_D, glayer0_W, glayer0_D,
           glayer1_W, glayer1_D, glayer2_W, glayer2_D, glayer3_W, glayer3_D,
           conv_out_W):
    lW = jnp.stack([layer0_W, layer1_W, layer2_W, layer3_W])
    lD = jnp.stack([layer0_D, layer1_D, layer2_D, layer3_D])
    gW = jnp.stack([glayer0_W, glayer1_W, glayer2_W, glayer3_W])
    gD = jnp.stack([glayer0_D, glayer1_D, glayer2_D, glayer3_D])
    xt = jnp.transpose(x, (0, 2, 1))

    rep = lambda s: pl.BlockSpec(s, lambda b: (0,) * len(s))
    zmean, z = pl.pallas_call(
        _fused_kernel,
        grid=(B,),
        in_specs=[
            pl.BlockSpec((1, 3, N), lambda b: (b, 0, 0)),
            pl.BlockSpec((1, N, 3), lambda b: (b, 0, 0)),
            rep((H, 3)),
            rep((H, H)),
            rep((NL, H, H)),
            rep((NL, H, H)),
            rep((NL, H, 2 * H)),
            rep((NL, H, H)),
            rep((CDIM, NL * H)),
        ],
        out_specs=[
            pl.BlockSpec((1, 3, CDIM), lambda b: (b, 0, 0)),
            pl.BlockSpec((1, 3, CDIM, N), lambda b: (b, 0, 0, 0)),
        ],
        out_shape=[
            jax.ShapeDtypeStruct((B, 3, CDIM), jnp.float32),
            jax.ShapeDtypeStruct((B, 3, CDIM, N), jnp.float32),
        ],
        scratch_shapes=[
            pltpu.VMEM((N, N), jnp.float32),
            pltpu.VMEM((3, H, N), jnp.float32),
        ],
    )(x, xt, conv_in_W, conv_in_D, lW, lD, gW, gD, conv_out_W)

    return (jnp.transpose(zmean, (0, 2, 1)), jnp.transpose(z, (0, 2, 1, 3)))


# speedup vs baseline: 1.0465x; 1.0465x over previous
import jax
import jax.numpy as jnp
from jax.experimental import pallas as pl
from jax.experimental.pallas import tpu as pltpu

H = 128
CDIM = 128
NL = 4
K = 16
B = 4
N = 1024
EPS = 1e-6
PB = 2


def _mm(W, v):
    return jnp.einsum('oi,vin->von', W.astype(jnp.bfloat16),
                      v.astype(jnp.bfloat16),
                      preferred_element_type=jnp.float32)


def _vact(v, D):
    d = _mm(D, v)
    dot = jnp.sum(v * d, axis=0, keepdims=True)
    dsq = jnp.sum(d * d, axis=0, keepdims=True)
    f = jnp.where(dot >= 0.0, 0.0, dot / (dsq + EPS))
    return v - f * d


def _fused_kernel(x_ref, xt_ref, ciW_ref, ciD_ref, lW_ref, lD_ref, gW_ref,
                  gD_ref, coW_ref, zmean_ref, z_ref, d2_ref, acc_ref):
    ciW = ciW_ref[...]
    ciD = ciD_ref[...]

    xbs = [x_ref[b] for b in range(PB)]
    xts = [xt_ref[b] for b in range(PB)]
    sqs, xdirs = [], []
    for b in range(PB):
        xb = xbs[b]
        sq = jnp.sum(xb * xb, axis=0)
        xb16 = xb.astype(jnp.bfloat16)
        gram = jax.lax.dot_general(xb16, xb16, (((0,), (0,)), ((), ())),
                                   preferred_element_type=jnp.float32)
        d2_ref[b] = sq[:, None] + sq[None, :] - 2.0 * gram
        sqs.append(sq)
        xdirs.append(xb / jnp.clip(jnp.sqrt(sq)[None, :], 1e-12, None))
        acc_ref[b] = jnp.zeros((3, H, N), jnp.float32)

    iota_j = jax.lax.broadcasted_iota(jnp.int32, (N, N), 0)

    def knn_body(k, _):
        for b in range(PB):
            xb = xbs[b]
            d2 = d2_ref[b]
            m = jnp.min(d2, axis=0, keepdims=True)
            cand = jnp.where(d2 == m, iota_j, N)
            amin = jnp.min(cand, axis=0, keepdims=True)
            sel = iota_j == amin
            d2_ref[b] = jnp.where(sel, jnp.float32(jnp.inf), d2)
            nbr = jnp.concatenate([
                jnp.sum(jnp.where(sel, xts[b][:, c:c + 1], 0.0), axis=0,
                        keepdims=True)
                for c in range(3)
            ], axis=0)
            rel = nbr - xb
            x_dir = xdirs[b]
            crs = jnp.stack([
                x_dir[1] * nbr[2] - x_dir[2] * nbr[1],
                x_dir[2] * nbr[0] - x_dir[0] * nbr[2],
                x_dir[0] * nbr[1] - x_dir[1] * nbr[0],
            ], axis=0)
            c16 = crs.astype(jnp.bfloat16).astype(jnp.float32)
            r16 = rel.astype(jnp.bfloat16).astype(jnp.float32)
            x16 = xb.astype(jnp.bfloat16).astype(jnp.float32)
            w16 = ciW.astype(jnp.bfloat16).astype(jnp.float32)
            h = (w16[None, :, 0:1] * c16[:, None, :] +
                 w16[None, :, 1:2] * r16[:, None, :] +
                 w16[None, :, 2:3] * x16[:, None, :])
            acc_ref[b] += _vact(h, ciD)
        return ()

    jax.lax.fori_loop(0, K, knn_body, (), unroll=2)

    for b in range(PB):
        y = acc_ref[b] * (1.0 / K)
        feats = []
        for i in range(NL):
            y = _vact(_mm(lW_ref[i], y), lD_ref[i])
            yg = jnp.mean(y, axis=2, keepdims=True)
            ycat = jnp.concatenate([y, jnp.broadcast_to(yg, y.shape)], axis=1)
            y = _vact(_mm(gW_ref[i], ycat), gD_ref[i])
            feats.append(y)
        z = jnp.concatenate(feats, axis=1)
        z = _mm(coW_ref[...], z)
        z_ref[b] = z
        zmean_ref[b] = jnp.mean(z, axis=2)


@jax.jit
def kernel(x, conv_in_W, conv_in_D, layer0_W, layer0_D, layer1_W, layer1_D,
           layer2_W, layer2_D, layer3_W, layer3_D, glayer0_W, glayer0_D,
           glayer1_W, glayer1_D, glayer2_W, glayer2_D, glayer3_W, glayer3_D,
           conv_out_W):
    lW = jnp.stack([layer0_W, layer1_W, layer2_W, layer3_W])
    lD = jnp.stack([layer0_D, layer1_D, layer2_D, layer3_D])
    gW = jnp.stack([glayer0_W, glayer1_W, glayer2_W, glayer3_W])
    gD = jnp.stack([glayer0_D, glayer1_D, glayer2_D, glayer3_D])
    xt = jnp.transpose(x, (0, 2, 1))

    rep = lambda s: pl.BlockSpec(s, lambda g: (0,) * len(s))
    zmean, z = pl.pallas_call(
        _fused_kernel,
        grid=(B // PB,),
        in_specs=[
            pl.BlockSpec((PB, 3, N), lambda g: (g, 0, 0)),
            pl.BlockSpec((PB, N, 3), lambda g: (g, 0, 0)),
            rep((H, 3)),
            rep((H, H)),
            rep((NL, H, H)),
            rep((NL, H, H)),
            rep((NL, H, 2 * H)),
            rep((NL, H, H)),
            rep((CDIM, NL * H)),
        ],
        out_specs=[
            pl.BlockSpec((PB, 3, CDIM), lambda g: (g, 0, 0)),
            pl.BlockSpec((PB, 3, CDIM, N), lambda g: (g, 0, 0, 0)),
        ],
        out_shape=[
            jax.ShapeDtypeStruct((B, 3, CDIM), jnp.float32),
            jax.ShapeDtypeStruct((B, 3, CDIM, N), jnp.float32),
        ],
        scratch_shapes=[
            pltpu.VMEM((PB, N, N), jnp.float32),
            pltpu.VMEM((PB, 3, H, N), jnp.float32),
        ],
    )(x, xt, conv_in_W, conv_in_D, lW, lD, gW, gD, conv_out_W)

    return (jnp.transpose(zmean, (0, 2, 1)), jnp.transpose(z, (0, 2, 1, 3)))


# speedup vs baseline: 1.2036x; 1.1501x over previous
import jax
import jax.numpy as jnp
from jax.experimental import pallas as pl
from jax.experimental.pallas import tpu as pltpu

H = 128
CDIM = 128
NL = 4
K = 16
B = 4
N = 1024
EPS = 1e-6


def _mm(W, v):
    return jnp.einsum('oi,vin->von', W.astype(jnp.bfloat16),
                      v.astype(jnp.bfloat16),
                      preferred_element_type=jnp.float32)


def _vact(v, D):
    d = _mm(D, v)
    dot = jnp.sum(v * d, axis=0, keepdims=True)
    dsq = jnp.sum(d * d, axis=0, keepdims=True)
    f = jnp.where(dot >= 0.0, 0.0, dot / (dsq + EPS))
    return v - f * d


def _fused_kernel(x_ref, xt_ref, ciW_ref, ciD_ref, lW_ref, lD_ref, gW_ref,
                  gD_ref, coW_ref, zmean_ref, z_ref, d2_ref, acc_ref):
    xb = x_ref[0]
    xt = xt_ref[0]

    sq = jnp.sum(xb * xb, axis=0)
    xb16 = xb.astype(jnp.bfloat16)
    gram = jax.lax.dot_general(xb16, xb16, (((0,), (0,)), ((), ())),
                               preferred_element_type=jnp.float32)
    d2_ref[...] = sq[:, None] + sq[None, :] - 2.0 * gram

    x_dir = xb / jnp.clip(jnp.sqrt(sq)[None, :], 1e-12, None)

    iota_j = jax.lax.broadcasted_iota(jnp.int32, (N, N), 0)
    ciW = ciW_ref[...]
    ciD = ciD_ref[...]
    acc_ref[...] = jnp.zeros((3, H, N), jnp.float32)

    def knn_body(k, _):
        d2 = d2_ref[...]
        m = jnp.min(d2, axis=0, keepdims=True)
        cand = jnp.where(d2 == m, iota_j, N)
        amin = jnp.min(cand, axis=0, keepdims=True)
        sel = iota_j == amin
        d2_ref[...] = jnp.where(sel, jnp.float32(jnp.inf), d2)
        nbr = jnp.concatenate([
            jnp.sum(jnp.where(sel, xt[:, c:c + 1], 0.0), axis=0, keepdims=True)
            for c in range(3)
        ], axis=0)
        rel = nbr - xb
        crs = jnp.stack([
            x_dir[1] * nbr[2] - x_dir[2] * nbr[1],
            x_dir[2] * nbr[0] - x_dir[0] * nbr[2],
            x_dir[0] * nbr[1] - x_dir[1] * nbr[0],
        ], axis=0)
        c16 = crs.astype(jnp.bfloat16).astype(jnp.float32)
        r16 = rel.astype(jnp.bfloat16).astype(jnp.float32)
        x16 = xb.astype(jnp.bfloat16).astype(jnp.float32)
        w16 = ciW.astype(jnp.bfloat16).astype(jnp.float32)
        h = (w16[None, :, 0:1] * c16[:, None, :] +
             w16[None, :, 1:2] * r16[:, None, :] +
             w16[None, :, 2:3] * x16[:, None, :])
        acc_ref[...] += _vact(h, ciD)
        return ()

    jax.lax.fori_loop(0, K, knn_body, (), unroll=4)

    y = acc_ref[...] * (1.0 / K)

    feats = []
    for i in range(NL):
        y = _vact(_mm(lW_ref[i], y), lD_ref[i])
        yg = jnp.mean(y, axis=2, keepdims=True)
        ycat = jnp.concatenate([y, jnp.broadcast_to(yg, y.shape)], axis=1)
        y = _vact(_mm(gW_ref[i], ycat), gD_ref[i])
        feats.append(y)

    z = jnp.concatenate(feats, axis=1)
    z = _mm(coW_ref[...], z)
    z_ref[0] = z
    zmean_ref[0] = jnp.mean(z, axis=2)


@jax.jit
def kernel(x, conv_in_W, conv_in_D, layer0_W, layer0_D, layer1_W, layer1_D,
           layer2_W, layer2_D, layer3_W, layer3_D, glayer0_W, glayer0_D,
           glayer1_W, glayer1_D, glayer2_W, glayer2_D, glayer3_W, glayer3_D,
           conv_out_W):
    lW = jnp.stack([layer0_W, layer1_W, layer2_W, layer3_W])
    lD = jnp.stack([layer0_D, layer1_D, layer2_D, layer3_D])
    gW = jnp.stack([glayer0_W, glayer1_W, glayer2_W, glayer3_W])
    gD = jnp.stack([glayer0_D, glayer1_D, glayer2_D, glayer3_D])
    xt = jnp.transpose(x, (0, 2, 1))

    rep = lambda s: pl.BlockSpec(s, lambda b: (0,) * len(s))
    zmean, z = pl.pallas_call(
        _fused_kernel,
        grid=(B,),
        in_specs=[
            pl.BlockSpec((1, 3, N), lambda b: (b, 0, 0)),
            pl.BlockSpec((1, N, 3), lambda b: (b, 0, 0)),
            rep((H, 3)),
            rep((H, H)),
            rep((NL, H, H)),
            rep((NL, H, H)),
            rep((NL, H, 2 * H)),
            rep((NL, H, H)),
            rep((CDIM, NL * H)),
        ],
        out_specs=[
            pl.BlockSpec((1, 3, CDIM), lambda b: (b, 0, 0)),
            pl.BlockSpec((1, 3, CDIM, N), lambda b: (b, 0, 0, 0)),
        ],
        out_shape=[
            jax.ShapeDtypeStruct((B, 3, CDIM), jnp.float32),
            jax.ShapeDtypeStruct((B, 3, CDIM, N), jnp.float32),
        ],
        scratch_shapes=[
            pltpu.VMEM((N, N), jnp.float32),
            pltpu.VMEM((3, H, N), jnp.float32),
        ],
    )(x, xt, conv_in_W, conv_in_D, lW, lD, gW, gD, conv_out_W)

    return (jnp.transpose(zmean, (0, 2, 1)), jnp.transpose(z, (0, 2, 1, 3)))
